# baseline (speedup 1.0000x reference)
import functools

import jax
import jax.numpy as jnp
from jax import lax
from jax.experimental import pallas as pl
from jax.experimental.pallas import tpu as pltpu
from jax.experimental.pallas import tpu_sc as plsc

B = 16384
D = 32
NC = 2
NS = 16
NW = NC * NS
BPW = B // NW
CH = 128
NCH = BPW // CH

WPB = 4
BLK = WPB * BPW


def _gather_body(utT, itT, uid3, iid3, out, idx, rows, sem):
    wid = lax.axis_index("s") * NC + lax.axis_index("c")
    for t, (tab, ids) in enumerate(((utT, uid3), (itT, iid3))):
        pltpu.sync_copy(ids.at[wid], idx)
        copies = []
        for j in range(NCH):
            for d in range(D):
                copies.append(
                    pltpu.async_copy(tab.at[d].at[idx.at[j]],
                                     rows.at[d, j], sem))
        for c in copies:
            c.wait()
        pltpu.sync_copy(rows, out.at[wid, t])


def _sc_gather(utT, itT, uid3, iid3):
    mesh = plsc.VectorSubcoreMesh(core_axis_name="c", subcore_axis_name="s")
    fn = functools.partial(
        pl.kernel,
        mesh=mesh,
        out_type=jax.ShapeDtypeStruct((NW, 2, D, NCH, CH), jnp.float32),
        scratch_types=[
            pltpu.VMEM((NCH, CH), jnp.int32),
            pltpu.VMEM((D, NCH, CH), jnp.float32),
            pltpu.SemaphoreType.DMA,
        ],
        compiler_params=pltpu.CompilerParams(use_tc_tiling_on_sc=False),
    )(_gather_body)
    return fn(utT, itT, uid3, iid3)


def _mlp_body(f, w1t, b1, w2t, b2, w3, b3, o):
    for w in range(WPB):
        for j in range(NCH):
            u = f[w, 0, :, j]
            i = f[w, 1, :, j]
            x = jnp.concatenate([u, i], axis=0)
            h = jnp.maximum(
                jnp.dot(w1t[...], x, preferred_element_type=jnp.float32)
                + b1[...], 0.0)
            h = jnp.maximum(
                jnp.dot(w2t[...], h, preferred_element_type=jnp.float32)
                + b2[...], 0.0)
            z = jnp.sum(h * w3[...], axis=0) + b3[0, 0]
            o[pl.ds(w * BPW + j * CH, CH)] = jax.nn.sigmoid(z)


def _tc_mlp(feats, W1, b1, W2, b2, W3, b3):
    w1t = W1.T
    w2t = W2.T
    b1c = b1.reshape(-1, 1)
    b2c = b2.reshape(-1, 1)
    b3r = b3.reshape(1, 1)
    grid = (NW // WPB,)
    return pl.pallas_call(
        _mlp_body,
        grid=grid,
        in_specs=[
            pl.BlockSpec((WPB, 2, D, NCH, CH), lambda g: (g, 0, 0, 0, 0)),
            pl.BlockSpec(w1t.shape, lambda g: (0, 0)),
            pl.BlockSpec(b1c.shape, lambda g: (0, 0)),
            pl.BlockSpec(w2t.shape, lambda g: (0, 0)),
            pl.BlockSpec(b2c.shape, lambda g: (0, 0)),
            pl.BlockSpec(W3.shape, lambda g: (0, 0)),
            pl.BlockSpec(memory_space=pltpu.SMEM),
        ],
        out_specs=pl.BlockSpec((BLK,), lambda g: (g,)),
        out_shape=jax.ShapeDtypeStruct((B,), jnp.float32),
    )(feats, w1t, b1c, w2t, b2c, W3, b3r)


def kernel(user_table, item_table, W1, b1, W2, b2, W3, b3, user_ids, item_ids):
    utT = user_table.T
    itT = item_table.T
    uid3 = user_ids.astype(jnp.int32).reshape(NW, NCH, CH)
    iid3 = item_ids.astype(jnp.int32).reshape(NW, NCH, CH)
    feats = _sc_gather(utT, itT, uid3, iid3)
    return _tc_mlp(feats, W1, b1, W2, b2, W3, b3)


# speedup vs baseline: 16.7964x; 16.7964x over previous
import functools

import jax
import jax.numpy as jnp
from jax import lax
from jax.experimental import pallas as pl
from jax.experimental.pallas import tpu as pltpu
from jax.experimental.pallas import tpu_sc as plsc

B = 16384
D = 32
PK = 4
DW = D * PK
NC = 2
NS = 16
NW = NC * NS
BPW = B // NW
CH = 128
NCH = BPW // CH

QU = 256000
QI = 25600
PROWS_U = 10240
PROWS_I = 1024

BLK = 2048


def _pack_body(x0, x1, x2, x3, o):
    X = jnp.concatenate([x0[...], x1[...], x2[...], x3[...]], axis=0)
    eye = (lax.broadcasted_iota(jnp.int32, (DW, DW), 0)
           == lax.broadcasted_iota(jnp.int32, (DW, DW), 1)).astype(jnp.float32)
    o[...] = lax.dot_general(X, eye, (((0,), (0,)), ((), ())),
                             preferred_element_type=jnp.float32)


def _pack(tT, q, prows):
    steps = q // prows
    maxb = (tT.shape[1] + prows - 1) // prows - 1
    in_specs = [
        pl.BlockSpec((D, prows),
                     lambda g, k=k: (0, jnp.minimum(k * steps + g, maxb)))
        for k in range(PK)
    ]
    return pl.pallas_call(
        _pack_body,
        grid=(steps,),
        in_specs=in_specs,
        out_specs=pl.BlockSpec((prows, DW), lambda g: (g, 0)),
        out_shape=jax.ShapeDtypeStruct((q, DW), jnp.float32),
    )(tT, tT, tT, tT)


def _gather_body(ut4, it4, uid3, iid3, u_out, i_out, idx, rows, sem):
    wid = lax.axis_index("s") * NC + lax.axis_index("c")
    base = wid * BPW
    for tab, ids, out in ((ut4, uid3, u_out), (it4, iid3, i_out)):
        pltpu.sync_copy(ids.at[wid], idx)
        copies = [
            pltpu.async_copy(tab.at[idx.at[j]], rows.at[j], sem)
            for j in range(NCH)
        ]
        for c in copies:
            c.wait()
        for j in range(NCH):
            pltpu.sync_copy(rows.at[j], out.at[pl.ds(base + j * CH, CH)])


def _sc_gather(ut4, it4, uid3, iid3):
    mesh = plsc.VectorSubcoreMesh(core_axis_name="c", subcore_axis_name="s")
    fn = functools.partial(
        pl.kernel,
        mesh=mesh,
        out_type=(
            jax.ShapeDtypeStruct((B, DW), jnp.float32),
            jax.ShapeDtypeStruct((B, DW), jnp.float32),
        ),
        scratch_types=[
            pltpu.VMEM((NCH, CH), jnp.int32),
            pltpu.VMEM((NCH, CH, DW), jnp.float32),
            pltpu.SemaphoreType.DMA,
        ],
    )(_gather_body)
    return fn(ut4, it4, uid3, iid3)


def _mlp_body(u, i, ulo, ilo, w1, b1, w2, b2, w3t, b3, o):
    lgrp = lax.broadcasted_iota(jnp.int32, (BLK, DW), 1) // D
    pick = (lax.broadcasted_iota(jnp.int32, (DW, D), 0) % D
            == lax.broadcasted_iota(jnp.int32, (DW, D), 1)).astype(jnp.float32)
    um = jnp.where(lgrp == ulo[...].reshape(BLK, 1), u[...], 0.0)
    im = jnp.where(lgrp == ilo[...].reshape(BLK, 1), i[...], 0.0)
    ue = jnp.dot(um, pick, preferred_element_type=jnp.float32)
    ie = jnp.dot(im, pick, preferred_element_type=jnp.float32)
    x = jnp.concatenate([ue, ie], axis=1)
    h = jnp.maximum(
        jnp.dot(x, w1[...], preferred_element_type=jnp.float32) + b1[...], 0.0)
    h = jnp.maximum(
        jnp.dot(h, w2[...], preferred_element_type=jnp.float32) + b2[...], 0.0)
    z = jnp.sum(h * w3t[...], axis=1) + b3[0, 0]
    o[...] = jax.nn.sigmoid(z)


def _tc_mlp(u_raw, i_raw, u_lo, i_lo, W1, b1, W2, b2, W3, b3):
    b1r = b1.reshape(1, -1)
    b2r = b2.reshape(1, -1)
    w3t = W3.reshape(1, -1)
    b3r = b3.reshape(1, 1)
    grid = (B // BLK,)
    return pl.pallas_call(
        _mlp_body,
        grid=grid,
        in_specs=[
            pl.BlockSpec((BLK, DW), lambda g: (g, 0)),
            pl.BlockSpec((BLK, DW), lambda g: (g, 0)),
            pl.BlockSpec((BLK,), lambda g: (g,)),
            pl.BlockSpec((BLK,), lambda g: (g,)),
            pl.BlockSpec(W1.shape, lambda g: (0, 0)),
            pl.BlockSpec(b1r.shape, lambda g: (0, 0)),
            pl.BlockSpec(W2.shape, lambda g: (0, 0)),
            pl.BlockSpec(b2r.shape, lambda g: (0, 0)),
            pl.BlockSpec(w3t.shape, lambda g: (0, 0)),
            pl.BlockSpec(memory_space=pltpu.SMEM),
        ],
        out_specs=pl.BlockSpec((BLK,), lambda g: (g,)),
        out_shape=jax.ShapeDtypeStruct((B,), jnp.float32),
    )(u_raw, i_raw, u_lo, i_lo, W1, b1r, W2, b2r, w3t, b3r)


def kernel(user_table, item_table, W1, b1, W2, b2, W3, b3, user_ids, item_ids):
    uids = user_ids.astype(jnp.int32)
    iids = item_ids.astype(jnp.int32)
    ut4 = _pack(user_table.T, QU, PROWS_U)
    it4 = _pack(item_table.T, QI, PROWS_I)
    uid3 = (uids % QU).reshape(NW, NCH, CH)
    iid3 = (iids % QI).reshape(NW, NCH, CH)
    u_lo = uids // QU
    i_lo = iids // QI
    u_raw, i_raw = _sc_gather(ut4, it4, uid3, iid3)
    return _tc_mlp(u_raw, i_raw, u_lo, i_lo, W1, b1, W2, b2, W3, b3)


# speedup vs baseline: 18.2991x; 1.0895x over previous
import functools

import jax
import jax.numpy as jnp
from jax import lax
from jax.experimental import pallas as pl
from jax.experimental.pallas import tpu as pltpu
from jax.experimental.pallas import tpu_sc as plsc

B = 16384
D = 32
PK = 4
DW = D * PK
NC = 2
NS = 16
NW = NC * NS
BPW = B // NW
CH = 128
NCH = BPW // CH

QU = 256000
QI = 25600
PROWS_U = 10240
PROWS_I = 5120

BLK = 4096


def _pack_body(x0, x1, x2, x3, o):
    X = jnp.concatenate([x0[...], x1[...], x2[...], x3[...]], axis=0)
    eye = (lax.broadcasted_iota(jnp.int32, (DW, DW), 0)
           == lax.broadcasted_iota(jnp.int32, (DW, DW), 1)).astype(jnp.float32)
    o[...] = lax.dot_general(X, eye, (((0,), (0,)), ((), ())),
                             preferred_element_type=jnp.float32)


def _pack(tT, q, prows):
    steps = q // prows
    maxb = (tT.shape[1] + prows - 1) // prows - 1
    in_specs = [
        pl.BlockSpec((D, prows),
                     lambda g, k=k: (0, jnp.minimum(k * steps + g, maxb)))
        for k in range(PK)
    ]
    return pl.pallas_call(
        _pack_body,
        grid=(steps,),
        in_specs=in_specs,
        out_specs=pl.BlockSpec((prows, DW), lambda g: (g, 0)),
        out_shape=jax.ShapeDtypeStruct((q, DW), jnp.float32),
    )(tT, tT, tT, tT)


def _gather_body(tab, ids, out, idx, rows, sem):
    wid = lax.axis_index("s") * NC + lax.axis_index("c")
    base = wid * BPW
    pltpu.sync_copy(ids.at[wid], idx)
    copies = [
        pltpu.async_copy(tab.at[idx.at[j]], rows.at[j], sem)
        for j in range(NCH)
    ]
    for c in copies:
        c.wait()
    for j in range(NCH):
        pltpu.sync_copy(rows.at[j], out.at[pl.ds(base + j * CH, CH)])


def _sc_gather(tab, id3):
    mesh = plsc.VectorSubcoreMesh(core_axis_name="c", subcore_axis_name="s")
    fn = functools.partial(
        pl.kernel,
        mesh=mesh,
        out_type=jax.ShapeDtypeStruct((B, DW), jnp.float32),
        scratch_types=[
            pltpu.VMEM((NCH, CH), jnp.int32),
            pltpu.VMEM((NCH, CH, DW), jnp.float32),
            pltpu.SemaphoreType.DMA,
        ],
    )(_gather_body)
    return fn(tab, id3)


def _mlp_body(u, i, ulo, ilo, w1, b1, w2, b2, w3t, b3, o):
    lgrp = lax.broadcasted_iota(jnp.int32, (BLK, DW), 1) // D
    pick = (lax.broadcasted_iota(jnp.int32, (DW, D), 0) % D
            == lax.broadcasted_iota(jnp.int32, (DW, D), 1)).astype(jnp.float32)
    um = jnp.where(lgrp == ulo[...].reshape(BLK, 1), u[...], 0.0)
    im = jnp.where(lgrp == ilo[...].reshape(BLK, 1), i[...], 0.0)
    ue = jnp.dot(um, pick, preferred_element_type=jnp.float32)
    ie = jnp.dot(im, pick, preferred_element_type=jnp.float32)
    x = jnp.concatenate([ue, ie], axis=1)
    h = jnp.maximum(
        jnp.dot(x, w1[...], preferred_element_type=jnp.float32) + b1[...], 0.0)
    h = jnp.maximum(
        jnp.dot(h, w2[...], preferred_element_type=jnp.float32) + b2[...], 0.0)
    z = jnp.sum(h * w3t[...], axis=1) + b3[0, 0]
    o[...] = jax.nn.sigmoid(z)


def _tc_mlp(u_raw, i_raw, u_lo, i_lo, W1, b1, W2, b2, W3, b3):
    b1r = b1.reshape(1, -1)
    b2r = b2.reshape(1, -1)
    w3t = W3.reshape(1, -1)
    b3r = b3.reshape(1, 1)
    grid = (B // BLK,)
    return pl.pallas_call(
        _mlp_body,
        grid=grid,
        in_specs=[
            pl.BlockSpec((BLK, DW), lambda g: (g, 0)),
            pl.BlockSpec((BLK, DW), lambda g: (g, 0)),
            pl.BlockSpec((BLK,), lambda g: (g,)),
            pl.BlockSpec((BLK,), lambda g: (g,)),
            pl.BlockSpec(W1.shape, lambda g: (0, 0)),
            pl.BlockSpec(b1r.shape, lambda g: (0, 0)),
            pl.BlockSpec(W2.shape, lambda g: (0, 0)),
            pl.BlockSpec(b2r.shape, lambda g: (0, 0)),
            pl.BlockSpec(w3t.shape, lambda g: (0, 0)),
            pl.BlockSpec(memory_space=pltpu.SMEM),
        ],
        out_specs=pl.BlockSpec((BLK,), lambda g: (g,)),
        out_shape=jax.ShapeDtypeStruct((B,), jnp.float32),
    )(u_raw, i_raw, u_lo, i_lo, W1, b1r, W2, b2r, w3t, b3r)


def kernel(user_table, item_table, W1, b1, W2, b2, W3, b3, user_ids, item_ids):
    uids = user_ids.astype(jnp.int32)
    iids = item_ids.astype(jnp.int32)
    uid3 = (uids % QU).reshape(NW, NCH, CH)
    iid3 = (iids % QI).reshape(NW, NCH, CH)
    u_lo = uids // QU
    i_lo = iids // QI
    it4 = _pack(item_table.T, QI, PROWS_I)
    i_raw = _sc_gather(it4, iid3)
    ut4 = _pack(user_table.T, QU, PROWS_U)
    u_raw = _sc_gather(ut4, uid3)
    return _tc_mlp(u_raw, i_raw, u_lo, i_lo, W1, b1, W2, b2, W3, b3)


# speedup vs baseline: 20.3671x; 1.1130x over previous
import functools

import jax
import jax.numpy as jnp
from jax import lax
from jax.experimental import pallas as pl
from jax.experimental.pallas import tpu as pltpu
from jax.experimental.pallas import tpu_sc as plsc

B = 16384
D = 32
PK = 8
DW = 128
NC = 2
NS = 16
NW = NC * NS
BPW = B // NW
CH = 128
NCH = BPW // CH

QU = 128000
QI = 12800
PROWS_U = 5120
PROWS_I = 2560

BLK = 4096


def _pack_body(*refs):
    xs = refs[:PK]
    o = refs[PK]
    X = jnp.concatenate([x[...] for x in xs], axis=0)
    n = PK * D
    eye = (lax.broadcasted_iota(jnp.int32, (n, n), 0)
           == lax.broadcasted_iota(jnp.int32, (n, n), 1)).astype(jnp.float32)
    y = lax.dot_general(X, eye, (((0,), (0,)), ((), ())),
                        preferred_element_type=jnp.float32)
    lo_bits = lax.bitcast_convert_type(y[:, 0:DW], jnp.uint32)
    hi_bits = lax.bitcast_convert_type(y[:, DW:2 * DW], jnp.uint32)
    lo16 = (lo_bits + jnp.uint32(0x8000)) >> 16
    hi16 = (hi_bits + jnp.uint32(0x8000)) & jnp.uint32(0xFFFF0000)
    o[...] = lax.bitcast_convert_type(hi16 | lo16, jnp.int32)


def _pack(tT, q, prows):
    steps = q // prows
    maxb = (tT.shape[1] + prows - 1) // prows - 1
    in_specs = [
        pl.BlockSpec((D, prows),
                     lambda g, k=k: (0, jnp.minimum(k * steps + g, maxb)))
        for k in range(PK)
    ]
    return pl.pallas_call(
        _pack_body,
        grid=(steps,),
        in_specs=in_specs,
        out_specs=pl.BlockSpec((prows, DW), lambda g: (g, 0)),
        out_shape=jax.ShapeDtypeStruct((q, DW), jnp.int32),
    )(*([tT] * PK))


def _gather_body(tab, ids, out, idx, rows, sem):
    wid = lax.axis_index("s") * NC + lax.axis_index("c")
    base = wid * BPW
    pltpu.sync_copy(ids.at[wid], idx)
    copies = [
        pltpu.async_copy(tab.at[idx.at[j]], rows.at[j], sem)
        for j in range(NCH)
    ]
    for c in copies:
        c.wait()
    for j in range(NCH):
        pltpu.sync_copy(rows.at[j], out.at[pl.ds(base + j * CH, CH)])


def _sc_gather(tab, id3):
    mesh = plsc.VectorSubcoreMesh(core_axis_name="c", subcore_axis_name="s")
    fn = functools.partial(
        pl.kernel,
        mesh=mesh,
        out_type=jax.ShapeDtypeStruct((B, DW), jnp.int32),
        scratch_types=[
            pltpu.VMEM((NCH, CH), jnp.int32),
            pltpu.VMEM((NCH, CH, DW), jnp.int32),
            pltpu.SemaphoreType.DMA,
        ],
    )(_gather_body)
    return fn(tab, id3)


def _mlp_body(u, i, ulo, ilo, w1, b1, w2, b2, w3t, b3, o):
    lgrp = lax.broadcasted_iota(jnp.int32, (BLK, DW), 1) // D
    pick = (lax.broadcasted_iota(jnp.int32, (DW, D), 0) % D
            == lax.broadcasted_iota(jnp.int32, (DW, D), 1)).astype(jnp.float32)

    def select(raw, lo):
        k = lo[...].reshape(BLK, 1)
        bits = lax.bitcast_convert_type(raw[...], jnp.uint32)
        lowf = lax.bitcast_convert_type(bits << 16, jnp.float32)
        highf = lax.bitcast_convert_type(
            bits & jnp.uint32(0xFFFF0000), jnp.float32)
        chosen = jnp.where(k < 4, lowf, highf)
        masked = jnp.where(lgrp == k % 4, chosen, 0.0)
        return jnp.dot(masked, pick, preferred_element_type=jnp.float32)

    x = jnp.concatenate([select(u, ulo), select(i, ilo)], axis=1)
    h = jnp.maximum(
        jnp.dot(x, w1[...], preferred_element_type=jnp.float32) + b1[...], 0.0)
    h = jnp.maximum(
        jnp.dot(h, w2[...], preferred_element_type=jnp.float32) + b2[...], 0.0)
    z = jnp.sum(h * w3t[...], axis=1) + b3[0, 0]
    o[...] = jax.nn.sigmoid(z)


def _tc_mlp(u_raw, i_raw, u_lo, i_lo, W1, b1, W2, b2, W3, b3):
    b1r = b1.reshape(1, -1)
    b2r = b2.reshape(1, -1)
    w3t = W3.reshape(1, -1)
    b3r = b3.reshape(1, 1)
    grid = (B // BLK,)
    return pl.pallas_call(
        _mlp_body,
        grid=grid,
        in_specs=[
            pl.BlockSpec((BLK, DW), lambda g: (g, 0)),
            pl.BlockSpec((BLK, DW), lambda g: (g, 0)),
            pl.BlockSpec((BLK,), lambda g: (g,)),
            pl.BlockSpec((BLK,), lambda g: (g,)),
            pl.BlockSpec(W1.shape, lambda g: (0, 0)),
            pl.BlockSpec(b1r.shape, lambda g: (0, 0)),
            pl.BlockSpec(W2.shape, lambda g: (0, 0)),
            pl.BlockSpec(b2r.shape, lambda g: (0, 0)),
            pl.BlockSpec(w3t.shape, lambda g: (0, 0)),
            pl.BlockSpec(memory_space=pltpu.SMEM),
        ],
        out_specs=pl.BlockSpec((BLK,), lambda g: (g,)),
        out_shape=jax.ShapeDtypeStruct((B,), jnp.float32),
    )(u_raw, i_raw, u_lo, i_lo, W1, b1r, W2, b2r, w3t, b3r)


def kernel(user_table, item_table, W1, b1, W2, b2, W3, b3, user_ids, item_ids):
    uids = user_ids.astype(jnp.int32)
    iids = item_ids.astype(jnp.int32)
    uid3 = (uids % QU).reshape(NW, NCH, CH)
    iid3 = (iids % QI).reshape(NW, NCH, CH)
    u_lo = uids // QU
    i_lo = iids // QI
    it4 = _pack(item_table.T, QI, PROWS_I)
    i_raw = _sc_gather(it4, iid3)
    ut4 = _pack(user_table.T, QU, PROWS_U)
    u_raw = _sc_gather(ut4, uid3)
    return _tc_mlp(u_raw, i_raw, u_lo, i_lo, W1, b1, W2, b2, W3, b3)
